# baseline (speedup 1.0000x reference)
import functools

import jax
import jax.numpy as jnp
from jax import lax
from jax.experimental import pallas as pl
from jax.experimental.pallas import tpu as pltpu
from jax.experimental.pallas import tpu_sc as plsc

BATCH = 16384
DIM = 64
NUM_CORES = 2
NUM_SUBCORES = 16
NUM_WORKERS = NUM_CORES * NUM_SUBCORES
ROWS_PER_WORKER = BATCH // NUM_WORKERS
CHUNK = 128
NCHUNK = ROWS_PER_WORKER // CHUNK
L = 16
NGROUP = ROWS_PER_WORKER // L


def _fused_body(table_hbm, lab_hbm, xt_hbm, w_hbm, out_hbm, idx_v, rows_v,
                xt_v, w_v, o_v, sem):
    wid = lax.axis_index("s") * NUM_CORES + lax.axis_index("c")
    base = wid * ROWS_PER_WORKER
    cbase = wid * NCHUNK
    pltpu.sync_copy(lab_hbm.at[pl.ds(cbase, NCHUNK)], idx_v)
    copies = [
        pltpu.async_copy(
            table_hbm.at[idx_v.at[j]], rows_v.at[pl.ds(j * CHUNK, CHUNK)], sem
        )
        for j in range(NCHUNK)
    ]
    pltpu.sync_copy(xt_hbm.at[:, pl.ds(base, ROWS_PER_WORKER)], xt_v)
    pltpu.sync_copy(w_hbm.at[pl.ds(base, ROWS_PER_WORKER)], w_v)
    for c in copies:
        c.wait()

    rows2d = rows_v
    iota = lax.iota(jnp.int32, L)

    def group(g, carry):
        j0 = g * L
        jv = j0 + iota
        zero = jnp.zeros((L,), jnp.float32)
        dot = zero
        na2 = zero
        nb2 = zero
        for c in range(DIM):
            dv = plsc.load_gather(rows2d, [jv, jnp.full((L,), c, jnp.int32)])
            xv = xt_v[c, pl.ds(j0, L)]
            dot = dot + dv * xv
            na2 = na2 + dv * dv
            nb2 = nb2 + xv * xv
        o_v[0, pl.ds(j0, L)] = dot * w_v[pl.ds(j0, L)]
        o_v[1, pl.ds(j0, L)] = na2
        o_v[2, pl.ds(j0, L)] = nb2
        return carry

    lax.fori_loop(0, NGROUP, group, 0)
    pltpu.sync_copy(o_v, out_hbm.at[:, pl.ds(base, ROWS_PER_WORKER)])


@jax.jit
def _sc_fused(table, lab2d, xt, w):
    mesh = plsc.VectorSubcoreMesh(core_axis_name="c", subcore_axis_name="s")
    f = functools.partial(
        pl.kernel,
        out_type=jax.ShapeDtypeStruct((3, BATCH), jnp.float32),
        mesh=mesh,
        scratch_types=[
            pltpu.VMEM((NCHUNK, CHUNK), jnp.int32),
            pltpu.VMEM((ROWS_PER_WORKER, DIM), jnp.float32),
            pltpu.VMEM((DIM, ROWS_PER_WORKER), jnp.float32),
            pltpu.VMEM((ROWS_PER_WORKER,), jnp.float32),
            pltpu.VMEM((3, ROWS_PER_WORKER), jnp.float32),
            pltpu.SemaphoreType.DMA,
        ],
        compiler_params=pltpu.CompilerParams(
            use_tc_tiling_on_sc=False, needs_layout_passes=False
        ),
    )(_fused_body)
    return f(table, lab2d, xt, w)


def _finish_body(o_ref, s_ref):
    dot = o_ref[0:1, :]
    na = jnp.maximum(jnp.sqrt(o_ref[1:2, :]), 1e-8)
    nb = jnp.maximum(jnp.sqrt(o_ref[2:3, :]), 1e-8)
    s_ref[...] = (jnp.sum(dot / (na * nb)) * (-1.0 / BATCH)).reshape(1, 1)


@jax.jit
def _tc_finish(out3):
    return pl.pallas_call(
        _finish_body,
        out_shape=jax.ShapeDtypeStruct((1, 1), jnp.float32),
    )(out3)


def kernel(x, label, weight, embedding_matrix):
    lab2d = label.astype(jnp.int32).reshape(BATCH // CHUNK, CHUNK)
    out3 = _sc_fused(embedding_matrix, lab2d, x.T, weight)
    return _tc_finish(out3)[0, 0]


# speedup vs baseline: 1.6151x; 1.6151x over previous
import functools

import jax
import jax.numpy as jnp
from jax import lax
from jax.experimental import pallas as pl
from jax.experimental.pallas import tpu as pltpu
from jax.experimental.pallas import tpu_sc as plsc

BATCH = 16384
DIM = 64
VOCAB = 100000
NUM_CORES = 2
NUM_SUBCORES = 16
DIMS_PER_WORKER = DIM // (NUM_CORES * NUM_SUBCORES)
QUARTER = BATCH // 4
L = 16
GROUPS = QUARTER // L


def _sc_body(tbl_hbm, xt_hbm, lab_hbm, out_hbm, row_v, xrow_v, lab_v, dot_v,
             sem):
    core = lax.axis_index("c")
    sid = lax.axis_index("s")

    for d in range(DIMS_PER_WORKER):
        r = core * (NUM_SUBCORES * DIMS_PER_WORKER) + sid * DIMS_PER_WORKER + d
        c_row = pltpu.async_copy(tbl_hbm.at[r >> 3, r & 7], row_v, sem)
        c_x = pltpu.async_copy(xt_hbm.at[r >> 3, r & 7], xrow_v, sem)
        c_row.wait()
        c_x.wait()
        for q in range(4):
            pltpu.sync_copy(lab_hbm.at[pl.ds(q * QUARTER, QUARTER)], lab_v)

            @plsc.parallel_loop(0, GROUPS, unroll=8)
            def group(j, q=q):
                j16 = j * L
                lv = lab_v[pl.ds(j16, L)]
                dv = plsc.load_gather(row_v, [lv])
                xv = xrow_v[pl.ds(q * QUARTER + j16, L)]
                dot_v[pl.ds(j16, L)] = dv * xv
                dot_v[pl.ds(QUARTER + j16, L)] = dv * dv

            pltpu.sync_copy(
                dot_v, out_hbm.at[r, pl.ds(q * 2 * QUARTER, 2 * QUARTER)])


@jax.jit
def _sc_dot(tbl3, xt3, lab):
    mesh = plsc.VectorSubcoreMesh(core_axis_name="c", subcore_axis_name="s")
    f = functools.partial(
        pl.kernel,
        out_type=jax.ShapeDtypeStruct((DIM, 2 * BATCH), jnp.float32),
        mesh=mesh,
        scratch_types=[
            pltpu.VMEM((VOCAB,), jnp.float32),
            pltpu.VMEM((BATCH,), jnp.float32),
            pltpu.VMEM((QUARTER,), jnp.int32),
            pltpu.VMEM((2 * QUARTER,), jnp.float32),
            pltpu.SemaphoreType.DMA,
        ],
        compiler_params=pltpu.CompilerParams(
            use_tc_tiling_on_sc=True, needs_layout_passes=False
        ),
    )(_sc_body)
    return f(tbl3, xt3, lab)


def _finish_body(p_ref, xt_ref, w_ref, s_ref):
    p = p_ref[...].reshape(DIM, 4, 2, QUARTER)
    dot = jnp.sum(p[:, :, 0, :], axis=0)
    na2 = jnp.sum(p[:, :, 1, :], axis=0)
    xt = xt_ref[...]
    nb2 = jnp.sum(xt * xt, axis=0).reshape(4, QUARTER)
    na = jnp.maximum(jnp.sqrt(na2), 1e-8)
    nb = jnp.maximum(jnp.sqrt(nb2), 1e-8)
    cos = dot / (na * nb)
    w = w_ref[...].reshape(4, QUARTER)
    s_ref[...] = (jnp.sum(cos * w) * (-1.0 / BATCH)).reshape(1, 1)


@jax.jit
def _tc_finish(part, xt, w2):
    return pl.pallas_call(
        _finish_body,
        out_shape=jax.ShapeDtypeStruct((1, 1), jnp.float32),
    )(part, xt, w2)


def kernel(x, label, weight, embedding_matrix):
    tbl3 = embedding_matrix.T.reshape(8, 8, VOCAB)
    xt = x.T
    xt3 = xt.reshape(8, 8, BATCH)
    lab = label.astype(jnp.int32)
    part = _sc_dot(tbl3, xt3, lab)
    return _tc_finish(part, xt, weight.reshape(1, BATCH))[0, 0]


# speedup vs baseline: 1.8291x; 1.1325x over previous
import functools

import jax
import jax.numpy as jnp
from jax import lax
from jax.experimental import pallas as pl
from jax.experimental.pallas import tpu as pltpu
from jax.experimental.pallas import tpu_sc as plsc

BATCH = 16384
DIM = 64
VOCAB = 100000
NUM_CORES = 2
NUM_SUBCORES = 16
DIMS_PER_WORKER = DIM // (NUM_CORES * NUM_SUBCORES)
QUARTER = BATCH // 4
L = 16
GROUPS = QUARTER // L


def _sc_body(tbl_hbm, xt_hbm, lab_hbm, out_hbm, row_v, xrow_v, lab_v, dot_v,
             na_v, sem):
    core = lax.axis_index("c")
    sid = lax.axis_index("s")

    for d in range(DIMS_PER_WORKER):
        r = core * (NUM_SUBCORES * DIMS_PER_WORKER) + sid * DIMS_PER_WORKER + d
        c_row = pltpu.async_copy(tbl_hbm.at[r >> 3, r & 7], row_v, sem)
        c_x = pltpu.async_copy(xt_hbm.at[r >> 3, r & 7], xrow_v, sem)
        c_row.wait()
        c_x.wait()
        for q in range(4):
            pltpu.sync_copy(lab_hbm.at[pl.ds(q * QUARTER, QUARTER)], lab_v)

            @plsc.parallel_loop(0, GROUPS, unroll=8)
            def group(j, q=q):
                j16 = j * L
                lv = lab_v[pl.ds(j16, L)]
                dv = plsc.load_gather(row_v, [lv])
                xv = xrow_v[pl.ds(q * QUARTER + j16, L)]
                dot_v[pl.ds(j16, L)] = dv * xv
                na_v[pl.ds(j16, L)] = dv * dv

            pltpu.sync_copy(dot_v, out_hbm.at[r, pl.ds(q * QUARTER, QUARTER)])
            pltpu.sync_copy(
                na_v, out_hbm.at[r, pl.ds(BATCH + q * QUARTER, QUARTER)])


@jax.jit
def _sc_dot(tbl3, xt3, lab):
    mesh = plsc.VectorSubcoreMesh(core_axis_name="c", subcore_axis_name="s")
    f = functools.partial(
        pl.kernel,
        out_type=jax.ShapeDtypeStruct((DIM, 2 * BATCH), jnp.float32),
        mesh=mesh,
        scratch_types=[
            pltpu.VMEM((VOCAB,), jnp.float32),
            pltpu.VMEM((BATCH,), jnp.float32),
            pltpu.VMEM((QUARTER,), jnp.int32),
            pltpu.VMEM((QUARTER,), jnp.float32),
            pltpu.VMEM((QUARTER,), jnp.float32),
            pltpu.SemaphoreType.DMA,
        ],
        compiler_params=pltpu.CompilerParams(
            use_tc_tiling_on_sc=True, needs_layout_passes=False
        ),
    )(_sc_body)
    return f(tbl3, xt3, lab)


def _finish_body(p_ref, xt_ref, w_ref, s_ref):
    dot = jnp.sum(p_ref[:, :BATCH], axis=0, keepdims=True)
    na2 = jnp.sum(p_ref[:, BATCH:], axis=0, keepdims=True)
    xt = xt_ref[...]
    nb2 = jnp.sum(xt * xt, axis=0, keepdims=True)
    na = jnp.maximum(jnp.sqrt(na2), 1e-8)
    nb = jnp.maximum(jnp.sqrt(nb2), 1e-8)
    cos = dot / (na * nb)
    s_ref[...] = (jnp.sum(cos * w_ref[...]) * (-1.0 / BATCH)).reshape(1, 1)


@jax.jit
def _tc_finish(part, xt, w2):
    return pl.pallas_call(
        _finish_body,
        out_shape=jax.ShapeDtypeStruct((1, 1), jnp.float32),
    )(part, xt, w2)


def kernel(x, label, weight, embedding_matrix):
    tbl3 = embedding_matrix.T.reshape(8, 8, VOCAB)
    xt = x.T
    xt3 = xt.reshape(8, 8, BATCH)
    lab = label.astype(jnp.int32)
    part = _sc_dot(tbl3, xt3, lab)
    return _tc_finish(part, xt, weight.reshape(1, BATCH))[0, 0]
